# baseline (speedup 1.0000x reference)
import functools

import jax
import jax.numpy as jnp
from jax import lax
from jax.experimental import pallas as pl
from jax.experimental.pallas import tpu as pltpu
from jax.experimental.pallas import tpu_sc as plsc

VOCAB = 100000
EMB = 64
UNITS = 64
BATCH = 1024
SEQ = 50
HSEQ = SEQ // 2

TBLK = 8192
HB = TBLK // 2
TGRID = -(-VOCAB // TBLK)
PAIRS = TGRID * HB


def _tr_body(tin_ref, out_ref):
    v = tin_ref[...]
    vt = v.T
    out_ref[...] = jnp.concatenate([vt[0:HB], vt[HB:TBLK]], axis=1)


def _tc_prep(tbl_t):
    return pl.pallas_call(
        _tr_body,
        grid=(TGRID,),
        in_specs=[pl.BlockSpec((EMB, TBLK), lambda i: (0, i))],
        out_specs=pl.BlockSpec((HB, 2 * EMB), lambda i: (i, 0)),
        out_shape=jax.ShapeDtypeStruct((PAIRS, 2 * EMB), jnp.float32),
    )(tbl_t)


NC = 2
NS = 16
NW = NC * NS
HALF_ROWS = BATCH * HSEQ
RW = HALF_ROWS // NW
CHUNK = 128
_CHUNKS = []
_off = 0
while _off < RW:
    _c = min(CHUNK, RW - _off)
    _CHUNKS.append((_off, _c))
    _off += _c


def _gather_body(table_hbm, idx_hbm, out_hbm, idx_v, rows_v, sem):
    wid = lax.axis_index("s") * NC + lax.axis_index("c")
    base = wid * RW
    pltpu.sync_copy(idx_hbm.at[pl.ds(base, RW)], idx_v)
    copies = []
    for off, c in _CHUNKS:
        cp = pltpu.async_copy(
            table_hbm.at[idx_v.at[pl.ds(off, c)]],
            rows_v.at[pl.ds(off, c)],
            sem,
        )
        copies.append(cp)
    for cp in copies:
        cp.wait()
    pltpu.sync_copy(rows_v, out_hbm.at[pl.ds(base, RW)])


def _sc_gather(table_pairs, idx):
    mesh = plsc.VectorSubcoreMesh(core_axis_name="c", subcore_axis_name="s")
    f = functools.partial(
        pl.kernel,
        mesh=mesh,
        out_type=jax.ShapeDtypeStruct((HALF_ROWS, 2 * EMB), jnp.float32),
        scratch_types=[
            pltpu.VMEM((RW,), jnp.int32),
            pltpu.VMEM((RW, 2 * EMB), jnp.float32),
            pltpu.SemaphoreType.DMA,
        ],
        compiler_params=pltpu.CompilerParams(use_tc_tiling_on_sc=True),
    )(_gather_body)
    return f(table_pairs, idx)


SPI = 5
HGRID = HSEQ // SPI


def _gru_common(emb_ref, par_ref, w_ref, u_ref, bt_ref, h0_ref,
                seq_ref, last_ref, embt_ref, h_ref):
    g = pl.program_id(0)

    @pl.when(g == 0)
    def _():
        h_ref[...] = h0_ref[...]

    W = w_ref[...]
    U = u_ref[...]
    bi = bt_ref[:, 0:1]
    br = bt_ref[:, 1:2]
    h = h_ref[...]

    for j in range(SPI):
        buf = emb_ref[j]
        par = par_ref[j] != 0
        buf_t = buf.T
        x_t = jnp.where(par, buf_t[EMB:2 * EMB], buf_t[0:EMB])
        xp = lax.dot_general(W, x_t, (((0,), (0,)), ((), ())),
                             preferred_element_type=jnp.float32) + bi
        hp = lax.dot_general(U, h, (((0,), (0,)), ((), ())),
                             preferred_element_type=jnp.float32) + br
        z = jax.nn.sigmoid(xp[0:UNITS] + hp[0:UNITS])
        r = jax.nn.sigmoid(xp[UNITS:2 * UNITS] + hp[UNITS:2 * UNITS])
        hh = jnp.tanh(xp[2 * UNITS:] + r * hp[2 * UNITS:])
        h = z * h + (1.0 - z) * hh
        seq_ref[j] = h
        embt_ref[j] = x_t

    h_ref[...] = h

    @pl.when(g == HGRID - 1)
    def _():
        last_ref[...] = h


def _gru_phase0_body(emb_ref, par_ref, w_ref, u_ref, bt_ref, h0_ref,
                     seq_ref, last_ref, embt_ref, h_ref):
    _gru_common(emb_ref, par_ref, w_ref, u_ref, bt_ref, h0_ref,
                seq_ref, last_ref, embt_ref, h_ref)


def _gru_phase1_body(emb_ref, par_ref, w_ref, u_ref, bt_ref, h0_ref,
                     seq_in_ref, embt_in_ref,
                     seq_ref, last_ref, embt_ref, h_ref):
    del seq_in_ref, embt_in_ref
    _gru_common(emb_ref, par_ref, w_ref, u_ref, bt_ref, h0_ref,
                seq_ref, last_ref, embt_ref, h_ref)


def _tc_gru(phase, emb128, par, W, U, bt, h0, seq_prev=None, embt_prev=None):
    off = phase * HGRID
    common_in = [
        pl.BlockSpec((SPI, BATCH, 2 * EMB), lambda s: (s, 0, 0)),
        pl.BlockSpec((SPI, 1, BATCH), lambda s: (s, 0, 0)),
        pl.BlockSpec((EMB, 3 * UNITS), lambda s: (0, 0)),
        pl.BlockSpec((UNITS, 3 * UNITS), lambda s: (0, 0)),
        pl.BlockSpec((3 * UNITS, 2), lambda s: (0, 0)),
        pl.BlockSpec((UNITS, BATCH), lambda s: (0, 0)),
    ]
    out_specs = [
        pl.BlockSpec((SPI, UNITS, BATCH), lambda s: (s + off, 0, 0)),
        pl.BlockSpec((UNITS, BATCH), lambda s: (0, 0)),
        pl.BlockSpec((SPI, EMB, BATCH), lambda s: (s + off, 0, 0)),
    ]
    out_shape = [
        jax.ShapeDtypeStruct((SEQ, UNITS, BATCH), jnp.float32),
        jax.ShapeDtypeStruct((UNITS, BATCH), jnp.float32),
        jax.ShapeDtypeStruct((SEQ, EMB, BATCH), jnp.float32),
    ]
    scratch = [pltpu.VMEM((UNITS, BATCH), jnp.float32)]
    if phase == 0:
        return pl.pallas_call(
            _gru_phase0_body,
            grid=(HGRID,),
            in_specs=common_in,
            out_specs=out_specs,
            out_shape=out_shape,
            scratch_shapes=scratch,
        )(emb128, par, W, U, bt, h0)
    hbm = pl.BlockSpec(memory_space=pltpu.MemorySpace.HBM)
    return pl.pallas_call(
        _gru_phase1_body,
        grid=(HGRID,),
        in_specs=common_in + [hbm, hbm],
        out_specs=out_specs,
        out_shape=out_shape,
        scratch_shapes=scratch,
        input_output_aliases={6: 0, 7: 2},
    )(emb128, par, W, U, bt, h0, seq_prev, embt_prev)


def kernel(x, initial, emb_table, W, U, b):
    del initial
    idx = jnp.swapaxes(x, 0, 1).reshape(-1).astype(jnp.int32)
    pair_idx = ((idx >> 13) << 12) | (idx & (HB - 1))
    par = ((idx >> 12) & 1).reshape(SEQ, 1, BATCH)
    table_pairs = _tc_prep(emb_table.T)
    rows0 = _sc_gather(table_pairs, pair_idx[:HALF_ROWS])
    rows1 = _sc_gather(table_pairs, pair_idx[HALF_ROWS:])
    emb0 = rows0.reshape(HSEQ, BATCH, 2 * EMB)
    emb1 = rows1.reshape(HSEQ, BATCH, 2 * EMB)
    bt = b.T
    h0 = jnp.zeros((UNITS, BATCH), jnp.float32)
    seq_a, h_mid, embt_a = _tc_gru(0, emb0, par[:HSEQ], W, U, bt, h0)
    seq_t, last_t, emb_t = _tc_gru(1, emb1, par[HSEQ:], W, U, bt, h_mid,
                                   seq_a, embt_a)
    seq_out = jnp.transpose(seq_t, (2, 0, 1))
    last_state = jnp.transpose(last_t, (1, 0))
    embedded = jnp.transpose(emb_t, (2, 0, 1))
    return (seq_out, last_state, embedded)
